# baseline (speedup 1.0000x reference)
import functools

import jax
import jax.numpy as jnp
from jax import lax
from jax.experimental import pallas as pl
from jax.experimental.pallas import tpu as pltpu
from jax.experimental.pallas import tpu_sc as plsc

EMB = 64
SCALE = 8.0

NC = 2
NS = 16
NW = NC * NS

CHUNK = 128
NBUF = 4
MUL_UNROLL = 8


def _build(num_tokens):
    per_w = num_tokens // NW
    ngroups = per_w // CHUNK
    iters = ngroups // NBUF
    mesh = plsc.VectorSubcoreMesh(core_axis_name="c", subcore_axis_name="s")

    @functools.partial(
        pl.kernel,
        out_type=jax.ShapeDtypeStruct((num_tokens, EMB), jnp.float32),
        mesh=mesh,
        compiler_params=pltpu.CompilerParams(use_tc_tiling_on_sc=False),
        scratch_types=(
            [pltpu.VMEM((ngroups, CHUNK), jnp.int32)]
            + [pltpu.VMEM((CHUNK, EMB), jnp.float32) for _ in range(NBUF)]
            + [pltpu.SemaphoreType.DMA for _ in range(2 * NBUF)]
        ),
    )
    def emb_kernel(tokens_hbm, table_hbm, out_hbm, idx_v, *rest):
        bufs = rest[:NBUF]
        gsems = rest[NBUF:2 * NBUF]
        osems = rest[2 * NBUF:]

        wid = lax.axis_index("s") * NC + lax.axis_index("c")
        base = wid * per_w

        pltpu.sync_copy(tokens_hbm.at[wid], idx_v)

        def gather(gi, b):
            pltpu.async_copy(table_hbm.at[idx_v.at[gi]], bufs[b], gsems[b])

        def gather_wait(gi, b):
            pltpu.make_async_copy(table_hbm.at[idx_v.at[gi]], bufs[b], gsems[b]).wait()

        def out_start(gi, b):
            pltpu.async_copy(bufs[b], out_hbm.at[pl.ds(base + gi * CHUNK, CHUNK)], osems[b])

        def out_wait(gi, b):
            pltpu.make_async_copy(bufs[b], out_hbm.at[pl.ds(base + gi * CHUNK, CHUNK)], osems[b]).wait()

        for b in range(NBUF):
            gather(b, b)

        def outer(it, carry):
            for b in range(NBUF):
                gi = it * NBUF + b
                gather_wait(gi, b)

                def mul_body(r, c, b=b):
                    for rr in range(MUL_UNROLL):
                        row = r * MUL_UNROLL + rr
                        for j in range(EMB // 16):
                            sl = pl.ds(j * 16, 16)
                            bufs[b][row, sl] = bufs[b][row, sl] * SCALE
                    return c
                lax.fori_loop(0, CHUNK // MUL_UNROLL, mul_body, 0)

                out_start(gi, b)

                bt = (b + 2) % NBUF
                @pl.when(jnp.logical_and(gi >= 2, gi <= ngroups - 3))
                def _(gi=gi, bt=bt):
                    out_wait(gi - 2, bt)
                    gather(gi + 2, bt)
            return carry

        lax.fori_loop(0, iters, outer, 0)

        for b in range(NBUF):
            out_wait(ngroups - NBUF + b, b)

    return emb_kernel


def kernel(tokens, table):
    num_tokens = tokens.size
    tokens3 = tokens.reshape(NW, num_tokens // (NW * CHUNK), CHUNK).astype(jnp.int32)
    out = _build(num_tokens)(tokens3, table)
    return out.reshape(tokens.shape + (EMB,))
